# baseline (speedup 1.0000x reference)
import functools

import jax
import jax.numpy as jnp
from jax import lax
from jax.experimental import pallas as pl
from jax.experimental.pallas import tpu as pltpu
from jax.experimental.pallas import tpu_sc as plsc

N = 10000
E = 160000
D = 256

NC = 2
NS = 16
NW = NC * NS
L = 16

CHUNK = 128
E_PAD = 163840
ROWS_ALL = E_PAD // CHUNK
RPW = ROWS_ALL // NW
BIGB = 8


def _mesh():
    return plsc.VectorSubcoreMesh(core_axis_name="c", subcore_axis_name="s")


HALF = N // NC
TRASH = HALF
RPT_A = ROWS_ALL // NS
NBLK_A = RPT_A // BIGB
HBINS = 10240
HSLICE = HBINS // NS


def _hist_phase(dst2d):
    @functools.partial(
        pl.kernel,
        out_type=jax.ShapeDtypeStruct((2 * N,), jnp.float32),
        mesh=_mesh(),
        compiler_params=pltpu.CompilerParams(use_tc_tiling_on_sc=False,
                                             needs_layout_passes=False),
        scratch_types=[
            pltpu.VMEM((BIGB, CHUNK), jnp.int32),
            pltpu.VMEM((HBINS,), jnp.float32),
            pltpu.VMEM((NS, HSLICE), jnp.float32),
            pltpu.VMEM_SHARED((NS, HBINS), jnp.float32),
        ],
    )
    def k(dst_hbm, deg_hbm, slab, hist, red, hists_sh):
        c = lax.axis_index("c")
        s = lax.axis_index("s")
        ones = jnp.full((L,), 1.0, jnp.float32)

        def zb(i, _):
            hist[pl.ds(i * L, L)] = jnp.full((L,), 0.0, jnp.float32)
            return 0
        lax.fori_loop(0, HBINS // L, zb, 0)

        def blk(b, _):
            row0 = c * (ROWS_ALL // NC) + s * (RPT_A // NC) + b * BIGB
            pltpu.sync_copy(dst_hbm.at[pl.ds(row0, BIGB)], slab)

            def win(w, _):
                r = w // (CHUNK // L)
                col = (w % (CHUNK // L)) * L
                v = slab[r, pl.ds(col, L)]
                plsc.addupdate_scatter(hist, [v], ones)
                return 0
            lax.fori_loop(0, BIGB * (CHUNK // L), win, 0)
            return 0
        lax.fori_loop(0, NBLK_A // NC, blk, 0)

        pltpu.sync_copy(hist, hists_sh.at[s])
        plsc.subcore_barrier()

        for t in range(NS):
            pltpu.sync_copy(hists_sh.at[t, pl.ds(s * HSLICE, HSLICE)],
                            red.at[t])

        def rw(w, _):
            acc = red[0, pl.ds(w * L, L)]
            for t in range(1, NS):
                acc = acc + red[t, pl.ds(w * L, L)]
            hist[pl.ds(w * L, L)] = acc
            return 0
        lax.fori_loop(0, HSLICE // L, rw, 0)

        @pl.when(s < 15)
        def _():
            pltpu.sync_copy(hist.at[pl.ds(0, HSLICE)],
                            deg_hbm.at[pl.ds(c * N + s * HSLICE, HSLICE)])

        @pl.when(s == 15)
        def _():
            pltpu.sync_copy(hist.at[pl.ds(0, 400)],
                            deg_hbm.at[pl.ds(c * N + 9600, 400)])

    return k(dst2d)


ACC_ROWS = HALF + 8
HROWS = RPT_A // 2
SELSZ = HROWS * CHUNK + 2 * CHUNK + L
DUMP0 = SELSZ - L


def _scatter_phase(src2d, dst2d, y):
    @functools.partial(
        pl.kernel,
        out_type=jax.ShapeDtypeStruct((N, D), jnp.float32),
        mesh=_mesh(),
        compiler_params=pltpu.CompilerParams(use_tc_tiling_on_sc=False,
                                             needs_layout_passes=False),
        scratch_types=[
            pltpu.VMEM((BIGB, CHUNK), jnp.int32),
            pltpu.VMEM((BIGB, CHUNK), jnp.int32),
            pltpu.VMEM((SELSZ,), jnp.int32),
            pltpu.VMEM((SELSZ,), jnp.int32),
            pltpu.VMEM((CHUNK, D), jnp.float32),
            pltpu.VMEM_SHARED((ACC_ROWS, D), jnp.float32),
            pltpu.SemaphoreType.DMA,
        ],
    )
    def k(src_hbm, dst_hbm, y_hbm, acc_hbm,
          sslab, dslab, sel_s, sel_d, rb0, acc_sh, semg0):
        c = lax.axis_index("c")
        s = lax.axis_index("s")
        lo = c * HALF
        hi = lo + HALF
        trash16 = jnp.full((L,), TRASH, jnp.int32)
        zero16 = jnp.full((L,), 0, jnp.int32)

        def zb(i, _):
            r = i // (D // L)
            col = (i % (D // L)) * L
            rb0[r, pl.ds(col, L)] = jnp.full((L,), 0.0, jnp.float32)
            return 0
        lax.fori_loop(0, L * (D // L), zb, 0)
        for k2 in range(19):
            pltpu.sync_copy(rb0.at[pl.ds(0, L)],
                            acc_sh.at[pl.ds(s * 313 + k2 * L, L)])
        pltpu.sync_copy(rb0.at[pl.ds(0, 9)],
                        acc_sh.at[pl.ds(s * 313 + 304, 9)])

        plsc.subcore_barrier()

        for half in range(2):
            def blk(b, cnt):
                row0 = s * RPT_A + half * HROWS + b * BIGB
                pltpu.sync_copy(src_hbm.at[pl.ds(row0, BIGB)], sslab)
                pltpu.sync_copy(dst_hbm.at[pl.ds(row0, BIGB)], dslab)

                def win(w, cnt):
                    r = w // (CHUNK // L)
                    col = (w % (CHUNK // L)) * L
                    vd = dslab[r, pl.ds(col, L)]
                    vs = sslab[r, pl.ds(col, L)]
                    ok = (vd >= lo) & (vd < hi)
                    oki = jnp.where(ok, jnp.int32(1), jnp.int32(0))
                    cum = plsc.cumsum(oki)
                    lane = lax.iota(jnp.int32, L)
                    pos = jnp.where(ok, cnt + cum - 1, DUMP0 + lane)
                    plsc.store_scatter(sel_s, [pos], vs)
                    plsc.store_scatter(sel_d, [pos], vd - lo)
                    return cnt + jnp.sum(oki)
                return lax.fori_loop(0, BIGB * (CHUNK // L), win, cnt)
            cnt = lax.fori_loop(0, HROWS // BIGB, blk, jnp.int32(0))

            for k2 in range(CHUNK // L):
                sel_s[pl.ds(cnt + k2 * L, L)] = zero16
                sel_d[pl.ds(cnt + k2 * L, L)] = trash16

            nchunks = (cnt + CHUNK - 1) // CHUNK

            def chunk(j, _):
                j0 = j * CHUNK
                g0 = pltpu.async_copy(
                    y_hbm.at[sel_s.at[pl.ds(j0, CHUNK)]], rb0, semg0)
                g0.wait()
                pltpu.sync_copy(rb0, acc_sh.at[sel_d.at[pl.ds(j0, CHUNK)]],
                                add=True)
                return 0
            lax.fori_loop(0, nchunks, chunk, 0)

        plsc.subcore_barrier()

        off = jnp.where(s < 15, s * 312, 4680)

        @pl.when(s < 15)
        def _():
            for k2 in range(2):
                pltpu.sync_copy(acc_sh.at[pl.ds(off + k2 * CHUNK, CHUNK)],
                                rb0)
                pltpu.sync_copy(
                    rb0, acc_hbm.at[pl.ds(lo + off + k2 * CHUNK, CHUNK)])
            pltpu.sync_copy(acc_sh.at[pl.ds(off + 256, 56)],
                            rb0.at[pl.ds(0, 56)])
            pltpu.sync_copy(rb0.at[pl.ds(0, 56)],
                            acc_hbm.at[pl.ds(lo + off + 256, 56)])

        @pl.when(s == 15)
        def _():
            for k2 in range(2):
                pltpu.sync_copy(acc_sh.at[pl.ds(off + k2 * CHUNK, CHUNK)],
                                rb0)
                pltpu.sync_copy(
                    rb0, acc_hbm.at[pl.ds(lo + off + k2 * CHUNK, CHUNK)])
            pltpu.sync_copy(acc_sh.at[pl.ds(off + 256, 64)],
                            rb0.at[pl.ds(0, 64)])
            pltpu.sync_copy(rb0.at[pl.ds(0, 64)],
                            acc_hbm.at[pl.ds(lo + off + 256, 64)])

    return k(src2d, dst2d, y)


_ROWS_B = 1000


def _dense_phase(x, W, deg0, deg1):
    def body(x_ref, w_ref, d0_ref, d1_ref, y_ref, dinv_ref):
        xw = jnp.dot(x_ref[...], w_ref[...],
                     preferred_element_type=jnp.float32)
        dv = lax.rsqrt(d0_ref[...] + d1_ref[...] + 1.0)
        y_ref[...] = xw * dv
        dinv_ref[...] = dv

    return pl.pallas_call(
        body,
        grid=(N // _ROWS_B,),
        in_specs=[
            pl.BlockSpec((_ROWS_B, D), lambda i: (i, 0)),
            pl.BlockSpec((D, D), lambda i: (0, 0)),
            pl.BlockSpec((_ROWS_B, 1), lambda i: (i, 0)),
            pl.BlockSpec((_ROWS_B, 1), lambda i: (i, 0)),
        ],
        out_specs=[
            pl.BlockSpec((_ROWS_B, D), lambda i: (i, 0)),
            pl.BlockSpec((_ROWS_B, 1), lambda i: (i, 0)),
        ],
        out_shape=[
            jax.ShapeDtypeStruct((N, D), jnp.float32),
            jax.ShapeDtypeStruct((N, 1), jnp.float32),
        ],
    )(x, W, deg0, deg1)


def _finish_phase(acc, y, dinv, b2d):
    def body(acc_ref, y_ref, dinv_ref, b_ref, out_ref):
        out_ref[...] = (dinv_ref[...] * (acc_ref[...] + y_ref[...])
                        + b_ref[...])

    return pl.pallas_call(
        body,
        grid=(N // _ROWS_B,),
        in_specs=[
            pl.BlockSpec((_ROWS_B, D), lambda i: (i, 0)),
            pl.BlockSpec((_ROWS_B, D), lambda i: (i, 0)),
            pl.BlockSpec((_ROWS_B, 1), lambda i: (i, 0)),
            pl.BlockSpec((1, D), lambda i: (0, 0)),
        ],
        out_specs=pl.BlockSpec((_ROWS_B, D), lambda i: (i, 0)),
        out_shape=jax.ShapeDtypeStruct((N, D), jnp.float32),
    )(acc, y, dinv, b2d)


def kernel(x, edge_index, W, b):
    src = edge_index[0]
    dst = edge_index[1]
    src2d = jnp.pad(src, (0, E_PAD - E)).reshape(ROWS_ALL, CHUNK)
    dst2d = jnp.pad(dst, (0, E_PAD - E),
                    constant_values=N).reshape(ROWS_ALL, CHUNK)

    degf = _hist_phase(dst2d)

    y, dinv = _dense_phase(x, W, degf[:N].reshape(N, 1),
                           degf[N:].reshape(N, 1))

    acc = _scatter_phase(src2d, dst2d, y)

    return _finish_phase(acc, y, dinv, b.reshape(1, D))


# speedup vs baseline: 1.0089x; 1.0089x over previous
import functools

import jax
import jax.numpy as jnp
from jax import lax
from jax.experimental import pallas as pl
from jax.experimental.pallas import tpu as pltpu
from jax.experimental.pallas import tpu_sc as plsc

N = 10000
E = 160000
D = 256

NC = 2
NS = 16
NW = NC * NS
L = 16

CHUNK = 128
E_PAD = 163840
ROWS_ALL = E_PAD // CHUNK
RPW = ROWS_ALL // NW
BIGB = 8


def _mesh():
    return plsc.VectorSubcoreMesh(core_axis_name="c", subcore_axis_name="s")


HALF = N // NC
TRASH = HALF
RPT_A = ROWS_ALL // NS
NBLK_A = RPT_A // BIGB
HBINS = 10240
HSLICE = HBINS // NS


def _hist_phase(dst2d):
    @functools.partial(
        pl.kernel,
        out_type=jax.ShapeDtypeStruct((2 * N,), jnp.float32),
        mesh=_mesh(),
        compiler_params=pltpu.CompilerParams(use_tc_tiling_on_sc=False,
                                             needs_layout_passes=False),
        scratch_types=[
            pltpu.VMEM((BIGB, CHUNK), jnp.int32),
            pltpu.VMEM((HBINS,), jnp.float32),
            pltpu.VMEM((NS, HSLICE), jnp.float32),
            pltpu.VMEM_SHARED((NS, HBINS), jnp.float32),
        ],
    )
    def k(dst_hbm, deg_hbm, slab, hist, red, hists_sh):
        c = lax.axis_index("c")
        s = lax.axis_index("s")
        ones = jnp.full((L,), 1.0, jnp.float32)

        def zb(i, _):
            hist[pl.ds(i * L, L)] = jnp.full((L,), 0.0, jnp.float32)
            return 0
        lax.fori_loop(0, HBINS // L, zb, 0)

        def blk(b, _):
            row0 = c * (ROWS_ALL // NC) + s * (RPT_A // NC) + b * BIGB
            pltpu.sync_copy(dst_hbm.at[pl.ds(row0, BIGB)], slab)

            def win(w, _):
                r = w // (CHUNK // L)
                col = (w % (CHUNK // L)) * L
                v = slab[r, pl.ds(col, L)]
                plsc.addupdate_scatter(hist, [v], ones)
                return 0
            lax.fori_loop(0, BIGB * (CHUNK // L), win, 0)
            return 0
        lax.fori_loop(0, NBLK_A // NC, blk, 0)

        pltpu.sync_copy(hist, hists_sh.at[s])
        plsc.subcore_barrier()

        for t in range(NS):
            pltpu.sync_copy(hists_sh.at[t, pl.ds(s * HSLICE, HSLICE)],
                            red.at[t])

        def rw(w, _):
            acc = red[0, pl.ds(w * L, L)]
            for t in range(1, NS):
                acc = acc + red[t, pl.ds(w * L, L)]
            hist[pl.ds(w * L, L)] = acc
            return 0
        lax.fori_loop(0, HSLICE // L, rw, 0)

        @pl.when(s < 15)
        def _():
            pltpu.sync_copy(hist.at[pl.ds(0, HSLICE)],
                            deg_hbm.at[pl.ds(c * N + s * HSLICE, HSLICE)])

        @pl.when(s == 15)
        def _():
            pltpu.sync_copy(hist.at[pl.ds(0, 400)],
                            deg_hbm.at[pl.ds(c * N + 9600, 400)])

    return k(dst2d)


ACC_ROWS = HALF + 8
HROWS = RPT_A // 2
SELSZ = HROWS * CHUNK + 2 * CHUNK + L
DUMP0 = SELSZ - L


def _scatter_phase(src2d, dst2d, y):
    @functools.partial(
        pl.kernel,
        out_type=jax.ShapeDtypeStruct((N, D), jnp.float32),
        mesh=_mesh(),
        compiler_params=pltpu.CompilerParams(use_tc_tiling_on_sc=False,
                                             needs_layout_passes=False),
        scratch_types=[
            pltpu.VMEM((BIGB, CHUNK), jnp.int32),
            pltpu.VMEM((BIGB, CHUNK), jnp.int32),
            pltpu.VMEM((SELSZ,), jnp.int32),
            pltpu.VMEM((SELSZ,), jnp.int32),
            pltpu.VMEM((CHUNK, D), jnp.float32),
            pltpu.VMEM_SHARED((ACC_ROWS, D), jnp.float32),
            pltpu.SemaphoreType.DMA,
        ],
    )
    def k(src_hbm, dst_hbm, y_hbm, acc_hbm,
          sslab, dslab, sel_s, sel_d, rb0, acc_sh, semg0):
        c = lax.axis_index("c")
        s = lax.axis_index("s")
        lo = c * HALF
        hi = lo + HALF
        trash16 = jnp.full((L,), TRASH, jnp.int32)
        zero16 = jnp.full((L,), 0, jnp.int32)

        def zb(i, _):
            r = i // (D // L)
            col = (i % (D // L)) * L
            rb0[r, pl.ds(col, L)] = jnp.full((L,), 0.0, jnp.float32)
            return 0
        lax.fori_loop(0, L * (D // L), zb, 0)
        for k2 in range(19):
            pltpu.sync_copy(rb0.at[pl.ds(0, L)],
                            acc_sh.at[pl.ds(s * 313 + k2 * L, L)])
        pltpu.sync_copy(rb0.at[pl.ds(0, 9)],
                        acc_sh.at[pl.ds(s * 313 + 304, 9)])

        plsc.subcore_barrier()

        for half in range(2):
            def blk(b, cnt):
                row0 = s * RPT_A + half * HROWS + b * BIGB
                pltpu.sync_copy(src_hbm.at[pl.ds(row0, BIGB)], sslab)
                pltpu.sync_copy(dst_hbm.at[pl.ds(row0, BIGB)], dslab)

                def win(w, cnt):
                    r = w // (CHUNK // L)
                    col = (w % (CHUNK // L)) * L
                    vd = dslab[r, pl.ds(col, L)]
                    vs = sslab[r, pl.ds(col, L)]
                    ok = (vd >= lo) & (vd < hi)
                    oki = jnp.where(ok, jnp.int32(1), jnp.int32(0))
                    cum = plsc.cumsum(oki)
                    lane = lax.iota(jnp.int32, L)
                    pos = jnp.where(ok, cnt + cum - 1, DUMP0 + lane)
                    plsc.store_scatter(sel_s, [pos], vs)
                    plsc.store_scatter(sel_d, [pos], vd - lo)
                    return cnt + jnp.sum(oki)
                return lax.fori_loop(0, BIGB * (CHUNK // L), win, cnt)
            cnt = lax.fori_loop(0, HROWS // BIGB, blk, jnp.int32(0))

            for k2 in range(CHUNK // L):
                sel_s[pl.ds(cnt + k2 * L, L)] = zero16
                sel_d[pl.ds(cnt + k2 * L, L)] = trash16

            nchunks = (cnt + CHUNK - 1) // CHUNK

            def chunk(j, _):
                j0 = j * CHUNK
                g0 = pltpu.async_copy(
                    y_hbm.at[sel_s.at[pl.ds(j0, CHUNK)]], rb0, semg0)
                g0.wait()
                pltpu.sync_copy(rb0, acc_sh.at[sel_d.at[pl.ds(j0, CHUNK)]],
                                add=True)
                return 0
            lax.fori_loop(0, nchunks, chunk, 0)

        plsc.subcore_barrier()

        off = jnp.where(s < 15, s * 312, 4680)

        @pl.when(s < 15)
        def _():
            for k2 in range(2):
                pltpu.sync_copy(acc_sh.at[pl.ds(off + k2 * CHUNK, CHUNK)],
                                rb0)
                pltpu.sync_copy(
                    rb0, acc_hbm.at[pl.ds(lo + off + k2 * CHUNK, CHUNK)])
            pltpu.sync_copy(acc_sh.at[pl.ds(off + 256, 56)],
                            rb0.at[pl.ds(0, 56)])
            pltpu.sync_copy(rb0.at[pl.ds(0, 56)],
                            acc_hbm.at[pl.ds(lo + off + 256, 56)])

        @pl.when(s == 15)
        def _():
            for k2 in range(2):
                pltpu.sync_copy(acc_sh.at[pl.ds(off + k2 * CHUNK, CHUNK)],
                                rb0)
                pltpu.sync_copy(
                    rb0, acc_hbm.at[pl.ds(lo + off + k2 * CHUNK, CHUNK)])
            pltpu.sync_copy(acc_sh.at[pl.ds(off + 256, 64)],
                            rb0.at[pl.ds(0, 64)])
            pltpu.sync_copy(rb0.at[pl.ds(0, 64)],
                            acc_hbm.at[pl.ds(lo + off + 256, 64)])

    return k(src2d, dst2d, y)


_ROWS_B = 1000


def _dense_phase(x, W, degf2d):
    def body(x_ref, w_ref, d0_ref, d1_ref, y_ref, dinv_ref):
        xw = jnp.dot(x_ref[...], w_ref[...],
                     preferred_element_type=jnp.float32)
        dv = lax.rsqrt(d0_ref[...] + d1_ref[...] + 1.0)
        y_ref[...] = xw * dv
        dinv_ref[...] = dv

    nblk = N // _ROWS_B
    return pl.pallas_call(
        body,
        grid=(nblk,),
        in_specs=[
            pl.BlockSpec((_ROWS_B, D), lambda i: (i, 0)),
            pl.BlockSpec((D, D), lambda i: (0, 0)),
            pl.BlockSpec((_ROWS_B, 1), lambda i: (i, 0)),
            pl.BlockSpec((_ROWS_B, 1), lambda i: (i + nblk, 0)),
        ],
        out_specs=[
            pl.BlockSpec((_ROWS_B, D), lambda i: (i, 0)),
            pl.BlockSpec((_ROWS_B, 1), lambda i: (i, 0)),
        ],
        out_shape=[
            jax.ShapeDtypeStruct((N, D), jnp.float32),
            jax.ShapeDtypeStruct((N, 1), jnp.float32),
        ],
    )(x, W, degf2d, degf2d)


def _finish_phase(acc, y, dinv, b2d):
    def body(acc_ref, y_ref, dinv_ref, b_ref, out_ref):
        out_ref[...] = (dinv_ref[...] * (acc_ref[...] + y_ref[...])
                        + b_ref[...])

    return pl.pallas_call(
        body,
        grid=(N // _ROWS_B,),
        in_specs=[
            pl.BlockSpec((_ROWS_B, D), lambda i: (i, 0)),
            pl.BlockSpec((_ROWS_B, D), lambda i: (i, 0)),
            pl.BlockSpec((_ROWS_B, 1), lambda i: (i, 0)),
            pl.BlockSpec((1, D), lambda i: (0, 0)),
        ],
        out_specs=pl.BlockSpec((_ROWS_B, D), lambda i: (i, 0)),
        out_shape=jax.ShapeDtypeStruct((N, D), jnp.float32),
    )(acc, y, dinv, b2d)


def kernel(x, edge_index, W, b):
    src = edge_index[0]
    dst = edge_index[1]
    src2d = jnp.pad(src, (0, E_PAD - E)).reshape(ROWS_ALL, CHUNK)
    dst2d = jnp.pad(dst, (0, E_PAD - E),
                    constant_values=N).reshape(ROWS_ALL, CHUNK)

    degf = _hist_phase(dst2d)

    y, dinv = _dense_phase(x, W, degf.reshape(2 * N, 1))

    acc = _scatter_phase(src2d, dst2d, y)

    return _finish_phase(acc, y, dinv, b.reshape(1, D))
